# baseline (speedup 1.0000x reference)
import functools

import jax
import jax.numpy as jnp
from jax import lax
from jax.experimental import pallas as pl
from jax.experimental.pallas import tpu as pltpu
from jax.experimental.pallas import tpu_sc as plsc

BATCH = 16384
EDIM = 64
L2RG = 1e-05
W_CF = 0.1

NC = 2
NS = 16
NW = NC * NS
BPW = BATCH // NW
WW = 16
NWIN = BPW // WW
SUBL = 8


def _sc_stats(uid, pos, neg, ue3, ie3):
    mesh = plsc.VectorSubcoreMesh(
        core_axis_name="c", subcore_axis_name="s", num_cores=NC, num_subcores=NS
    )
    out_t = tuple(
        jax.ShapeDtypeStruct((BATCH,), jnp.float32) for _ in range(7)
    )
    gbuf = lambda: pltpu.VMEM((WW, SUBL, EDIM), jnp.float32)
    sbuf = lambda: pltpu.VMEM((BPW,), jnp.float32)

    @functools.partial(
        pl.kernel,
        out_type=out_t,
        mesh=mesh,
        scratch_types=[
            pltpu.VMEM((BPW,), jnp.int32),
            pltpu.VMEM((BPW,), jnp.int32),
            pltpu.VMEM((BPW,), jnp.int32),
            pltpu.VMEM((SUBL, EDIM), jnp.float32),
            gbuf(), gbuf(),
            gbuf(), gbuf(),
            gbuf(), gbuf(),
            sbuf(), sbuf(), sbuf(), sbuf(), sbuf(), sbuf(), sbuf(),
            pltpu.SemaphoreType.DMA,
            pltpu.SemaphoreType.DMA,
        ],
        compiler_params=pltpu.CompilerParams(needs_layout_passes=False),
    )
    def k(uid_h, pos_h, neg_h, ue_h, ie_h,
          o_dp, o_dn, o_su, o_sp, o_sn, o_cp, o_cn,
          uidx, pidx, nidx, cbuf, gu0, gu1, gp0, gp1, gn0, gn1,
          b_dp, b_dn, b_su, b_sp, b_sn, b_cp, b_cn,
          sem0, sem1):
        wid = lax.axis_index("s") * NC + lax.axis_index("c")
        base = wid * BPW
        pltpu.sync_copy(uid_h.at[pl.ds(base, BPW)], uidx)
        pltpu.sync_copy(pos_h.at[pl.ds(base, BPW)], pidx)
        pltpu.sync_copy(neg_h.at[pl.ds(base, BPW)], nidx)
        pltpu.sync_copy(ie_h.at[pl.ds(0, SUBL), :], cbuf)

        gu = (gu0, gu1)
        gp = (gp0, gp1)
        gn = (gn0, gn1)
        sems = (sem0, sem1)

        def window_tiles(g):
            st = g * WW
            tv_u = uidx[pl.ds(st, WW)] >> 3
            tv_p = pidx[pl.ds(st, WW)] >> 3
            tv_n = nidx[pl.ds(st, WW)] >> 3
            return tv_u, tv_p, tv_n

        def descriptors(g, slot):
            tv_u, tv_p, tv_n = window_tiles(g)
            sem = sems[slot]
            for j in range(WW):
                yield pltpu.make_async_copy(
                    ue_h.at[pl.ds(tv_u[j] * SUBL, SUBL), :],
                    gu[slot].at[j], sem)
                yield pltpu.make_async_copy(
                    ie_h.at[pl.ds(tv_p[j] * SUBL, SUBL), :],
                    gp[slot].at[j], sem)
                yield pltpu.make_async_copy(
                    ie_h.at[pl.ds(tv_n[j] * SUBL, SUBL), :],
                    gn[slot].at[j], sem)

        def issue(g, slot):
            for d in descriptors(g, slot):
                d.start()

        def drain(g, slot):
            for d in descriptors(g, slot):
                d.wait()

        ju = lax.iota(jnp.int32, WW)
        last_lane = ju == (WW - 1)
        nq = EDIM // WW
        cq = [cbuf[0, pl.ds(kk * WW, WW)] for kk in range(nq)]

        def compute(g, slot):
            bu, bp, bn = gu[slot], gp[slot], gn[slot]
            st = g * WW
            iv_u = uidx[pl.ds(st, WW)]
            iv_p = pidx[pl.ds(st, WW)]
            iv_n = nidx[pl.ds(st, WW)]
            for j in range(WW):
                s_u = iv_u[j] & 7
                s_p = iv_p[j] & 7
                s_n = iv_n[j] & 7
                uq = [bu[j, s_u, pl.ds(kk * WW, WW)] for kk in range(nq)]
                pq = [bp[j, s_p, pl.ds(kk * WW, WW)] for kk in range(nq)]
                nnq = [bn[j, s_n, pl.ds(kk * WW, WW)] for kk in range(nq)]

                def dot4(a, b):
                    return ((a[0] * b[0] + a[1] * b[1])
                            + (a[2] * b[2] + a[3] * b[3]))

                rv = jnp.full((WW,), st + j, jnp.int32)

                def put(buf, vec):
                    plsc.store_scatter(buf, [rv], plsc.cumsum(vec),
                                       mask=last_lane)

                put(b_dp, dot4(uq, pq))
                put(b_dn, dot4(uq, nnq))
                put(b_su, dot4(uq, uq))
                put(b_sp, dot4(pq, pq))
                put(b_sn, dot4(nnq, nnq))
                put(b_cp, dot4(cq, pq))
                put(b_cn, dot4(cq, nnq))

        issue(0, 0)

        def body(t, carry):
            g0 = t * 2
            drain(g0, 0)
            issue(g0 + 1, 1)
            compute(g0, 0)
            g1 = g0 + 1
            drain(g1, 1)

            @pl.when(t < NWIN // 2 - 1)
            def _():
                issue(g1 + 1, 0)

            compute(g1, 1)
            return carry

        lax.fori_loop(0, NWIN // 2, body, 0)

        pltpu.sync_copy(b_dp, o_dp.at[pl.ds(base, BPW)])
        pltpu.sync_copy(b_dn, o_dn.at[pl.ds(base, BPW)])
        pltpu.sync_copy(b_su, o_su.at[pl.ds(base, BPW)])
        pltpu.sync_copy(b_sp, o_sp.at[pl.ds(base, BPW)])
        pltpu.sync_copy(b_sn, o_sn.at[pl.ds(base, BPW)])
        pltpu.sync_copy(b_cp, o_cp.at[pl.ds(base, BPW)])
        pltpu.sync_copy(b_cn, o_cn.at[pl.ds(base, BPW)])

    return k(uid, pos, neg, ue3, ie3)


def _tc_body(dp_ref, dn_ref, su_ref, sp_ref, sn_ref, cp_ref, cn_ref,
             c_ref, alpha_ref, out_ref):
    dp = dp_ref[...]
    dn = dn_ref[...]
    su = su_ref[...]
    sp = sp_ref[...]
    sn = sn_ref[...]
    cp = cp_ref[...]
    cn = cn_ref[...]
    c = c_ref[...]
    alpha = alpha_ref[0]
    eps = 1e-12

    def softplus(x):
        return jnp.maximum(x, 0.0) + jnp.log1p(jnp.exp(-jnp.abs(x)))

    s_bce = jnp.sum(softplus(-alpha * dp)) + jnp.sum(softplus(alpha * dn))
    s_l2 = jnp.sum(su + sp + sn)

    sc = jnp.sum(c * c)
    mc = jnp.maximum(jnp.sqrt(sc), eps)
    ncc = sc / (mc * mc)
    mp = jnp.maximum(jnp.sqrt(sp), eps)
    mn = jnp.maximum(jnp.sqrt(sn), eps)
    s_cf = (jnp.sum(ncc + sp / (mp * mp) - 2.0 * cp / (mc * mp))
            + jnp.sum(ncc + sn / (mn * mn) - 2.0 * cn / (mc * mn)))

    binv = jnp.float32(1.0 / BATCH)
    out_ref[0] = (s_bce * binv
                  + jnp.float32(L2RG) * s_l2 * binv
                  + jnp.float32(W_CF) * s_cf * (binv / EDIM))


def _tc_reduce(stats, c, alpha):
    out = pl.pallas_call(
        _tc_body,
        in_specs=[pl.BlockSpec((BATCH,), lambda: (0,)) for _ in range(7)]
        + [pl.BlockSpec((1, EDIM), lambda: (0, 0)),
           pl.BlockSpec(memory_space=pltpu.SMEM)],
        out_specs=pl.BlockSpec(memory_space=pltpu.SMEM),
        out_shape=jax.ShapeDtypeStruct((1,), jnp.float32),
    )(*stats, c, alpha)
    return out[0]


def kernel(uid, seq, nbr, pos, neg, user_embs, item_embs, user_bias, item_bias, alpha):
    uid = uid.astype(jnp.int32)
    pos = pos.astype(jnp.int32)
    neg = neg.astype(jnp.int32)
    stats = _sc_stats(uid, pos, neg, user_embs, item_embs)
    c = lax.slice(item_embs, (0, 0), (1, EDIM))
    alpha_arr = jnp.reshape(alpha.astype(jnp.float32), (1,))
    return _tc_reduce(stats, c, alpha_arr)


# speedup vs baseline: 1.0509x; 1.0509x over previous
import functools

import jax
import jax.numpy as jnp
from jax import lax
from jax.experimental import pallas as pl
from jax.experimental.pallas import tpu as pltpu
from jax.experimental.pallas import tpu_sc as plsc

BATCH = 16384
EDIM = 64
L2RG = 1e-05
W_CF = 0.1

NC = 2
NS = 16
NW = NC * NS
BPW = BATCH // NW
WW = 16
NWIN = BPW // WW
SUBL = 8


def _sc_stats(uid, pos, neg, ue3, ie3):
    mesh = plsc.VectorSubcoreMesh(
        core_axis_name="c", subcore_axis_name="s", num_cores=NC, num_subcores=NS
    )
    out_t = tuple(
        jax.ShapeDtypeStruct((BATCH,), jnp.float32) for _ in range(7)
    )
    gbuf = lambda: pltpu.VMEM((WW, EDIM), jnp.float32)
    sbuf = lambda: pltpu.VMEM((BPW,), jnp.float32)

    @functools.partial(
        pl.kernel,
        out_type=out_t,
        mesh=mesh,
        scratch_types=[
            pltpu.VMEM((BPW,), jnp.int32),
            pltpu.VMEM((BPW,), jnp.int32),
            pltpu.VMEM((BPW,), jnp.int32),
            pltpu.VMEM((1, EDIM), jnp.float32),
            gbuf(), gbuf(),
            gbuf(), gbuf(),
            gbuf(), gbuf(),
            sbuf(), sbuf(), sbuf(), sbuf(), sbuf(), sbuf(), sbuf(),
            pltpu.SemaphoreType.DMA,
            pltpu.SemaphoreType.DMA,
        ],
        compiler_params=pltpu.CompilerParams(needs_layout_passes=False),
    )
    def k(uid_h, pos_h, neg_h, ue_h, ie_h,
          o_dp, o_dn, o_su, o_sp, o_sn, o_cp, o_cn,
          uidx, pidx, nidx, cbuf, gu0, gu1, gp0, gp1, gn0, gn1,
          b_dp, b_dn, b_su, b_sp, b_sn, b_cp, b_cn,
          sem0, sem1):
        wid = lax.axis_index("s") * NC + lax.axis_index("c")
        base = wid * BPW
        pltpu.sync_copy(uid_h.at[pl.ds(base, BPW)], uidx)
        pltpu.sync_copy(pos_h.at[pl.ds(base, BPW)], pidx)
        pltpu.sync_copy(neg_h.at[pl.ds(base, BPW)], nidx)
        pltpu.sync_copy(ie_h.at[pl.ds(0, 1), :], cbuf)

        gu = (gu0, gu1)
        gp = (gp0, gp1)
        gn = (gn0, gn1)
        sems = (sem0, sem1)

        def descriptors(g, slot):
            st = g * WW
            iv_u = uidx[pl.ds(st, WW)]
            iv_p = pidx[pl.ds(st, WW)]
            iv_n = nidx[pl.ds(st, WW)]
            sem = sems[slot]
            for j in range(WW):
                yield pltpu.make_async_copy(
                    ue_h.at[pl.ds(iv_u[j], 1), :],
                    gu[slot].at[pl.ds(j, 1), :], sem)
                yield pltpu.make_async_copy(
                    ie_h.at[pl.ds(iv_p[j], 1), :],
                    gp[slot].at[pl.ds(j, 1), :], sem)
                yield pltpu.make_async_copy(
                    ie_h.at[pl.ds(iv_n[j], 1), :],
                    gn[slot].at[pl.ds(j, 1), :], sem)

        def issue(g, slot):
            for d in descriptors(g, slot):
                d.start()

        def drain(g, slot):
            for d in descriptors(g, slot):
                d.wait()

        ju = lax.iota(jnp.int32, WW)
        last_lane = ju == (WW - 1)
        nq = EDIM // WW
        cq = [cbuf[0, pl.ds(kk * WW, WW)] for kk in range(nq)]

        def compute(g, slot):
            bu, bp, bn = gu[slot], gp[slot], gn[slot]
            st = g * WW
            for j in range(WW):
                uq = [bu[j, pl.ds(kk * WW, WW)] for kk in range(nq)]
                pq = [bp[j, pl.ds(kk * WW, WW)] for kk in range(nq)]
                nnq = [bn[j, pl.ds(kk * WW, WW)] for kk in range(nq)]

                def dot4(a, b):
                    return ((a[0] * b[0] + a[1] * b[1])
                            + (a[2] * b[2] + a[3] * b[3]))

                rv = jnp.full((WW,), st + j, jnp.int32)

                def put(buf, vec):
                    plsc.store_scatter(buf, [rv], plsc.cumsum(vec),
                                       mask=last_lane)

                put(b_dp, dot4(uq, pq))
                put(b_dn, dot4(uq, nnq))
                put(b_su, dot4(uq, uq))
                put(b_sp, dot4(pq, pq))
                put(b_sn, dot4(nnq, nnq))
                put(b_cp, dot4(cq, pq))
                put(b_cn, dot4(cq, nnq))

        issue(0, 0)

        def body(t, carry):
            g0 = t * 2
            drain(g0, 0)
            issue(g0 + 1, 1)
            compute(g0, 0)
            g1 = g0 + 1
            drain(g1, 1)

            @pl.when(t < NWIN // 2 - 1)
            def _():
                issue(g1 + 1, 0)

            compute(g1, 1)
            return carry

        lax.fori_loop(0, NWIN // 2, body, 0)

        pltpu.sync_copy(b_dp, o_dp.at[pl.ds(base, BPW)])
        pltpu.sync_copy(b_dn, o_dn.at[pl.ds(base, BPW)])
        pltpu.sync_copy(b_su, o_su.at[pl.ds(base, BPW)])
        pltpu.sync_copy(b_sp, o_sp.at[pl.ds(base, BPW)])
        pltpu.sync_copy(b_sn, o_sn.at[pl.ds(base, BPW)])
        pltpu.sync_copy(b_cp, o_cp.at[pl.ds(base, BPW)])
        pltpu.sync_copy(b_cn, o_cn.at[pl.ds(base, BPW)])

    return k(uid, pos, neg, ue3, ie3)


def _tc_body(dp_ref, dn_ref, su_ref, sp_ref, sn_ref, cp_ref, cn_ref,
             c_ref, alpha_ref, out_ref):
    dp = dp_ref[...]
    dn = dn_ref[...]
    su = su_ref[...]
    sp = sp_ref[...]
    sn = sn_ref[...]
    cp = cp_ref[...]
    cn = cn_ref[...]
    c = c_ref[...]
    alpha = alpha_ref[0]
    eps = 1e-12

    def softplus(x):
        return jnp.maximum(x, 0.0) + jnp.log1p(jnp.exp(-jnp.abs(x)))

    s_bce = jnp.sum(softplus(-alpha * dp)) + jnp.sum(softplus(alpha * dn))
    s_l2 = jnp.sum(su + sp + sn)

    sc = jnp.sum(c * c)
    mc = jnp.maximum(jnp.sqrt(sc), eps)
    ncc = sc / (mc * mc)
    mp = jnp.maximum(jnp.sqrt(sp), eps)
    mn = jnp.maximum(jnp.sqrt(sn), eps)
    s_cf = (jnp.sum(ncc + sp / (mp * mp) - 2.0 * cp / (mc * mp))
            + jnp.sum(ncc + sn / (mn * mn) - 2.0 * cn / (mc * mn)))

    binv = jnp.float32(1.0 / BATCH)
    out_ref[0] = (s_bce * binv
                  + jnp.float32(L2RG) * s_l2 * binv
                  + jnp.float32(W_CF) * s_cf * (binv / EDIM))


def _tc_reduce(stats, c, alpha):
    out = pl.pallas_call(
        _tc_body,
        in_specs=[pl.BlockSpec((BATCH,), lambda: (0,)) for _ in range(7)]
        + [pl.BlockSpec((1, EDIM), lambda: (0, 0)),
           pl.BlockSpec(memory_space=pltpu.SMEM)],
        out_specs=pl.BlockSpec(memory_space=pltpu.SMEM),
        out_shape=jax.ShapeDtypeStruct((1,), jnp.float32),
    )(*stats, c, alpha)
    return out[0]


def kernel(uid, seq, nbr, pos, neg, user_embs, item_embs, user_bias, item_bias, alpha):
    uid = uid.astype(jnp.int32)
    pos = pos.astype(jnp.int32)
    neg = neg.astype(jnp.int32)
    stats = _sc_stats(uid, pos, neg, user_embs, item_embs)
    c = lax.slice(item_embs, (0, 0), (1, EDIM))
    alpha_arr = jnp.reshape(alpha.astype(jnp.float32), (1,))
    return _tc_reduce(stats, c, alpha_arr)


# speedup vs baseline: 1.5564x; 1.4810x over previous
import functools

import jax
import jax.numpy as jnp
from jax import lax
from jax.experimental import pallas as pl
from jax.experimental.pallas import tpu as pltpu
from jax.experimental.pallas import tpu_sc as plsc

BATCH = 16384
EDIM = 64
L2RG = 1e-05
W_CF = 0.1

NC = 2
NS = 16
NW = NC * NS
BPW = BATCH // NW
WW = 16
NWIN = BPW // WW
SUBL = 8


def _sc_stats(uid, pos, neg, ue3, ie3):
    mesh = plsc.VectorSubcoreMesh(
        core_axis_name="c", subcore_axis_name="s", num_cores=NC, num_subcores=NS
    )
    out_t = tuple(
        jax.ShapeDtypeStruct((BATCH,), jnp.float32) for _ in range(7)
    )
    gbuf = lambda: pltpu.VMEM((WW, EDIM), jnp.float32)
    sbuf = lambda: pltpu.VMEM((BPW,), jnp.float32)

    @functools.partial(
        pl.kernel,
        out_type=out_t,
        mesh=mesh,
        scratch_types=[
            pltpu.VMEM((BPW,), jnp.int32),
            pltpu.VMEM((BPW,), jnp.int32),
            pltpu.VMEM((BPW,), jnp.int32),
            pltpu.VMEM((1, EDIM), jnp.float32),
            gbuf(), gbuf(),
            gbuf(), gbuf(),
            gbuf(), gbuf(),
            sbuf(), sbuf(), sbuf(), sbuf(), sbuf(), sbuf(), sbuf(),
            pltpu.SemaphoreType.DMA,
            pltpu.SemaphoreType.DMA,
        ],
        compiler_params=pltpu.CompilerParams(needs_layout_passes=False),
    )
    def k(uid_h, pos_h, neg_h, ue_h, ie_h,
          o_dp, o_dn, o_su, o_sp, o_sn, o_cp, o_cn,
          uidx, pidx, nidx, cbuf, gu0, gu1, gp0, gp1, gn0, gn1,
          b_dp, b_dn, b_su, b_sp, b_sn, b_cp, b_cn,
          sem0, sem1):
        wid = lax.axis_index("s") * NC + lax.axis_index("c")
        base = wid * BPW
        pltpu.sync_copy(uid_h.at[pl.ds(base, BPW)], uidx)
        pltpu.sync_copy(pos_h.at[pl.ds(base, BPW)], pidx)
        pltpu.sync_copy(neg_h.at[pl.ds(base, BPW)], nidx)
        pltpu.sync_copy(ie_h.at[0, pl.ds(0, 1), :], cbuf)

        gu = (gu0, gu1)
        gp = (gp0, gp1)
        gn = (gn0, gn1)
        sems = (sem0, sem1)

        def descriptors(g, slot):
            st = g * WW
            iv_u = uidx[pl.ds(st, WW)]
            iv_p = pidx[pl.ds(st, WW)]
            iv_n = nidx[pl.ds(st, WW)]
            sem = sems[slot]
            for j in range(WW):
                yield pltpu.make_async_copy(
                    ue_h.at[iv_u[j] >> 3, pl.ds(iv_u[j] & 7, 1), :],
                    gu[slot].at[pl.ds(j, 1), :], sem)
                yield pltpu.make_async_copy(
                    ie_h.at[iv_p[j] >> 3, pl.ds(iv_p[j] & 7, 1), :],
                    gp[slot].at[pl.ds(j, 1), :], sem)
                yield pltpu.make_async_copy(
                    ie_h.at[iv_n[j] >> 3, pl.ds(iv_n[j] & 7, 1), :],
                    gn[slot].at[pl.ds(j, 1), :], sem)

        def issue(g, slot):
            for d in descriptors(g, slot):
                d.start()

        def drain(g, slot):
            for d in descriptors(g, slot):
                d.wait()

        ju = lax.iota(jnp.int32, WW)
        last_lane = ju == (WW - 1)
        nq = EDIM // WW
        cq = [cbuf[0, pl.ds(kk * WW, WW)] for kk in range(nq)]

        def compute(g, slot):
            bu, bp, bn = gu[slot], gp[slot], gn[slot]
            st = g * WW
            for j in range(WW):
                uq = [bu[j, pl.ds(kk * WW, WW)] for kk in range(nq)]
                pq = [bp[j, pl.ds(kk * WW, WW)] for kk in range(nq)]
                nnq = [bn[j, pl.ds(kk * WW, WW)] for kk in range(nq)]

                def dot4(a, b):
                    return ((a[0] * b[0] + a[1] * b[1])
                            + (a[2] * b[2] + a[3] * b[3]))

                rv = jnp.full((WW,), st + j, jnp.int32)

                def put(buf, vec):
                    plsc.store_scatter(buf, [rv], plsc.cumsum(vec),
                                       mask=last_lane)

                put(b_dp, dot4(uq, pq))
                put(b_dn, dot4(uq, nnq))
                put(b_su, dot4(uq, uq))
                put(b_sp, dot4(pq, pq))
                put(b_sn, dot4(nnq, nnq))
                put(b_cp, dot4(cq, pq))
                put(b_cn, dot4(cq, nnq))

        issue(0, 0)

        def body(t, carry):
            g0 = t * 2
            drain(g0, 0)
            issue(g0 + 1, 1)
            compute(g0, 0)
            g1 = g0 + 1
            drain(g1, 1)

            @pl.when(t < NWIN // 2 - 1)
            def _():
                issue(g1 + 1, 0)

            compute(g1, 1)
            return carry

        lax.fori_loop(0, NWIN // 2, body, 0)

        pltpu.sync_copy(b_dp, o_dp.at[pl.ds(base, BPW)])
        pltpu.sync_copy(b_dn, o_dn.at[pl.ds(base, BPW)])
        pltpu.sync_copy(b_su, o_su.at[pl.ds(base, BPW)])
        pltpu.sync_copy(b_sp, o_sp.at[pl.ds(base, BPW)])
        pltpu.sync_copy(b_sn, o_sn.at[pl.ds(base, BPW)])
        pltpu.sync_copy(b_cp, o_cp.at[pl.ds(base, BPW)])
        pltpu.sync_copy(b_cn, o_cn.at[pl.ds(base, BPW)])

    return k(uid, pos, neg, ue3, ie3)


def _tc_body(dp_ref, dn_ref, su_ref, sp_ref, sn_ref, cp_ref, cn_ref,
             c_ref, alpha_ref, out_ref):
    dp = dp_ref[...]
    dn = dn_ref[...]
    su = su_ref[...]
    sp = sp_ref[...]
    sn = sn_ref[...]
    cp = cp_ref[...]
    cn = cn_ref[...]
    c = c_ref[...]
    alpha = alpha_ref[0]
    eps = 1e-12

    def softplus(x):
        return jnp.maximum(x, 0.0) + jnp.log1p(jnp.exp(-jnp.abs(x)))

    s_bce = jnp.sum(softplus(-alpha * dp)) + jnp.sum(softplus(alpha * dn))
    s_l2 = jnp.sum(su + sp + sn)

    sc = jnp.sum(c * c)
    mc = jnp.maximum(jnp.sqrt(sc), eps)
    ncc = sc / (mc * mc)
    mp = jnp.maximum(jnp.sqrt(sp), eps)
    mn = jnp.maximum(jnp.sqrt(sn), eps)
    s_cf = (jnp.sum(ncc + sp / (mp * mp) - 2.0 * cp / (mc * mp))
            + jnp.sum(ncc + sn / (mn * mn) - 2.0 * cn / (mc * mn)))

    binv = jnp.float32(1.0 / BATCH)
    out_ref[0] = (s_bce * binv
                  + jnp.float32(L2RG) * s_l2 * binv
                  + jnp.float32(W_CF) * s_cf * (binv / EDIM))


def _tc_reduce(stats, c, alpha):
    out = pl.pallas_call(
        _tc_body,
        in_specs=[pl.BlockSpec((BATCH,), lambda: (0,)) for _ in range(7)]
        + [pl.BlockSpec((1, EDIM), lambda: (0, 0)),
           pl.BlockSpec(memory_space=pltpu.SMEM)],
        out_specs=pl.BlockSpec(memory_space=pltpu.SMEM),
        out_shape=jax.ShapeDtypeStruct((1,), jnp.float32),
    )(*stats, c, alpha)
    return out[0]


def kernel(uid, seq, nbr, pos, neg, user_embs, item_embs, user_bias, item_bias, alpha):
    uid = uid.astype(jnp.int32)
    pos = pos.astype(jnp.int32)
    neg = neg.astype(jnp.int32)
    ue3 = jnp.reshape(user_embs, (user_embs.shape[0] // SUBL, SUBL, EDIM))
    ie3 = jnp.reshape(item_embs, (item_embs.shape[0] // SUBL, SUBL, EDIM))
    stats = _sc_stats(uid, pos, neg, ue3, ie3)
    c = lax.slice(item_embs, (0, 0), (1, EDIM))
    alpha_arr = jnp.reshape(alpha.astype(jnp.float32), (1,))
    return _tc_reduce(stats, c, alpha_arr)
